# Initial kernel scaffold; baseline (speedup 1.0000x reference)
import functools

import jax
import jax.numpy as jnp
from jax import lax
from jax.experimental import pallas as pl
from jax.experimental.pallas import tpu as pltpu
from jax.experimental.pallas import tpu_sc as plsc

_NUM_CORES = 2
_NUM_SUBCORES = 16
_NUM_WORKERS = _NUM_CORES * _NUM_SUBCORES
_LANES = 16


def _chunk_size(e_per_w: int) -> int:
    best = 0
    for c in range(8, 129, 8):
        if e_per_w % c == 0:
            best = c
    if best == 0:
        raise ValueError(f"no valid chunk size for {e_per_w} edges per worker")
    return best


@functools.partial(jax.jit, static_argnames=("interpret",))
def _scores(x, src, dst, interpret=False):
    e = src.shape[0]
    d = x.shape[1]
    e_per_w = e // _NUM_WORKERS
    ch = _chunk_size(e_per_w)
    n_ch = e_per_w // ch
    n_vec = d // _LANES

    def body(x_hbm, src_hbm, dst_hbm, out_hbm,
             idx_u, idx_v, rows_u, rows_v, scores, sem_u, sem_v):
        wid = lax.axis_index("s") * _NUM_CORES + lax.axis_index("c")
        base = wid * e_per_w
        pltpu.sync_copy(src_hbm.at[pl.ds(base, e_per_w)], idx_u)
        pltpu.sync_copy(dst_hbm.at[pl.ds(base, e_per_w)], idx_v)

        def chunk(g, _):
            cu = pltpu.async_copy(x_hbm.at[idx_u.at[pl.ds(g * ch, ch)]],
                                  rows_u, sem_u)
            cv = pltpu.async_copy(x_hbm.at[idx_v.at[pl.ds(g * ch, ch)]],
                                  rows_v, sem_v)
            cu.wait()
            cv.wait()

            def edge(k, _):
                acc = (rows_u[k, pl.ds(0, _LANES)]
                       * rows_v[k, pl.ds(0, _LANES)])
                for j in range(1, n_vec):
                    acc += (rows_u[k, pl.ds(j * _LANES, _LANES)]
                            * rows_v[k, pl.ds(j * _LANES, _LANES)])
                scores[g * ch + k] = jnp.sum(acc)
                return ()

            lax.fori_loop(0, ch, edge, (), unroll=4)
            return ()

        lax.fori_loop(0, n_ch, chunk, ())
        pltpu.sync_copy(scores, out_hbm.at[pl.ds(base, e_per_w)])

    mesh = plsc.VectorSubcoreMesh(core_axis_name="c", subcore_axis_name="s")
    return pl.kernel(
        body,
        out_type=jax.ShapeDtypeStruct((e,), jnp.float32),
        mesh=mesh,
        scratch_types=[
            pltpu.VMEM((e_per_w,), jnp.int32),
            pltpu.VMEM((e_per_w,), jnp.int32),
            pltpu.VMEM((ch, d), jnp.float32),
            pltpu.VMEM((ch, d), jnp.float32),
            pltpu.VMEM((e_per_w,), jnp.float32),
            pltpu.SemaphoreType.DMA,
            pltpu.SemaphoreType.DMA,
        ],
        interpret=interpret,
    )(x, src, dst)


def kernel(x, edge_index):
    src = edge_index[0].astype(jnp.int32)
    dst = edge_index[1].astype(jnp.int32)
    return _scores(x, src, dst)[:, None]


# speedup vs baseline: 1.4411x; 1.4411x over previous
import functools

import jax
import jax.numpy as jnp
from jax import lax
from jax.experimental import pallas as pl
from jax.experimental.pallas import tpu as pltpu
from jax.experimental.pallas import tpu_sc as plsc

_NUM_CORES = 2
_NUM_SUBCORES = 16
_NUM_WORKERS = _NUM_CORES * _NUM_SUBCORES
_LANES = 16
_CHUNK = 128


def _lane_take(v, idx):
    return lax.gather(
        v, idx[:, None],
        dimension_numbers=lax.GatherDimensionNumbers(
            offset_dims=(), collapsed_slice_dims=(0,), start_index_map=(0,)),
        slice_sizes=(1,),
        mode=lax.GatherScatterMode.PROMISE_IN_BOUNDS)


@functools.partial(jax.jit, static_argnames=("interpret",))
def _scores(x, src, dst, interpret=False):
    e_pad = src.shape[0]
    d = x.shape[1]
    e_per_w = e_pad // _NUM_WORKERS
    n_ch = e_per_w // _CHUNK
    n_grp = _CHUNK // _LANES
    n_vec = d // _LANES

    def body(x_hbm, src_hbm, dst_hbm, out_hbm,
             idx_u, idx_v, rows_u, rows_v, scores, sem_u, sem_v):
        wid = lax.axis_index("s") * _NUM_CORES + lax.axis_index("c")
        base = wid * e_per_w
        pltpu.sync_copy(src_hbm.at[pl.ds(base, e_per_w)], idx_u)
        pltpu.sync_copy(dst_hbm.at[pl.ds(base, e_per_w)], idx_v)
        lane = lax.broadcasted_iota(jnp.int32, (_LANES,), 0)

        def chunk(g, _):
            cu = pltpu.async_copy(x_hbm.at[idx_u.at[pl.ds(g * _CHUNK, _CHUNK)]],
                                  rows_u, sem_u)
            cv = pltpu.async_copy(x_hbm.at[idx_v.at[pl.ds(g * _CHUNK, _CHUNK)]],
                                  rows_v, sem_v)
            cu.wait()
            cv.wait()

            def group(t, _):
                def edge(k, sv):
                    e = t * _LANES + k
                    acc = (rows_u[e, pl.ds(0, _LANES)]
                           * rows_v[e, pl.ds(0, _LANES)])
                    for j in range(1, n_vec):
                        acc += (rows_u[e, pl.ds(j * _LANES, _LANES)]
                                * rows_v[e, pl.ds(j * _LANES, _LANES)])
                    for s in (1, 2, 4, 8):
                        acc = acc + _lane_take(acc, lane ^ s)
                    return jnp.where(lane == k, acc, sv)

                sv = lax.fori_loop(0, _LANES, edge,
                                   jnp.zeros((_LANES,), jnp.float32))
                scores[pl.ds(g * _CHUNK + t * _LANES, _LANES)] = sv
                return ()

            lax.fori_loop(0, n_grp, group, ())
            return ()

        lax.fori_loop(0, n_ch, chunk, ())
        pltpu.sync_copy(scores, out_hbm.at[pl.ds(base, e_per_w)])

    mesh = plsc.VectorSubcoreMesh(core_axis_name="c", subcore_axis_name="s",
                                  num_cores=_NUM_CORES,
                                  num_subcores=_NUM_SUBCORES)
    return pl.kernel(
        body,
        out_type=jax.ShapeDtypeStruct((e_pad,), jnp.float32),
        mesh=mesh,
        scratch_types=[
            pltpu.VMEM((e_per_w,), jnp.int32),
            pltpu.VMEM((e_per_w,), jnp.int32),
            pltpu.VMEM((_CHUNK, d), jnp.float32),
            pltpu.VMEM((_CHUNK, d), jnp.float32),
            pltpu.VMEM((e_per_w,), jnp.float32),
            pltpu.SemaphoreType.DMA,
            pltpu.SemaphoreType.DMA,
        ],
        interpret=interpret,
    )(x, src, dst)


def kernel(x, edge_index):
    e = edge_index.shape[1]
    quantum = _NUM_WORKERS * _CHUNK
    e_pad = ((e + quantum - 1) // quantum) * quantum
    src = edge_index[0].astype(jnp.int32)
    dst = edge_index[1].astype(jnp.int32)
    if e_pad != e:
        pad = jnp.zeros((e_pad - e,), jnp.int32)
        src = jnp.concatenate([src, pad])
        dst = jnp.concatenate([dst, pad])
    return _scores(x, src, dst)[:e, None]


# speedup vs baseline: 1.6795x; 1.1654x over previous
import functools

import jax
import jax.numpy as jnp
from jax import lax
from jax.experimental import pallas as pl
from jax.experimental.pallas import tpu as pltpu
from jax.experimental.pallas import tpu_sc as plsc

_NUM_CORES = 2
_NUM_SUBCORES = 16
_NUM_WORKERS = _NUM_CORES * _NUM_SUBCORES
_LANES = 16
_CHUNK = 80


def _lane_take(v, idx):
    return lax.gather(
        v, idx[:, None],
        dimension_numbers=lax.GatherDimensionNumbers(
            offset_dims=(), collapsed_slice_dims=(0,), start_index_map=(0,)),
        slice_sizes=(1,),
        mode=lax.GatherScatterMode.PROMISE_IN_BOUNDS)


@functools.partial(jax.jit, static_argnames=("interpret",))
def _scores(x, src, dst, interpret=False):
    e_pad = src.shape[0]
    d = x.shape[1]
    e_per_w = e_pad // _NUM_WORKERS
    n_ch = e_per_w // _CHUNK
    n_half = n_ch // 2
    n_grp = _CHUNK // _LANES
    n_vec = d // _LANES

    def body(x_hbm, src_hbm, dst_hbm, out_hbm,
             idx_u, idx_v, rows_u0, rows_v0, rows_u1, rows_v1, scores,
             sem_u0, sem_v0, sem_u1, sem_v1):
        wid = lax.axis_index("s") * _NUM_CORES + lax.axis_index("c")
        base = wid * e_per_w
        pltpu.sync_copy(src_hbm.at[pl.ds(base, e_per_w)], idx_u)
        pltpu.sync_copy(dst_hbm.at[pl.ds(base, e_per_w)], idx_v)
        lane = lax.broadcasted_iota(jnp.int32, (_LANES,), 0)

        def issue(g, bu, bv, su, sv):
            pltpu.async_copy(x_hbm.at[idx_u.at[pl.ds(g * _CHUNK, _CHUNK)]],
                             bu, su)
            pltpu.async_copy(x_hbm.at[idx_v.at[pl.ds(g * _CHUNK, _CHUNK)]],
                             bv, sv)

        def wait(bu, bv, su, sv):
            pltpu.make_async_copy(
                x_hbm.at[idx_u.at[pl.ds(0, _CHUNK)]], bu, su).wait()
            pltpu.make_async_copy(
                x_hbm.at[idx_v.at[pl.ds(0, _CHUNK)]], bv, sv).wait()

        def compute(g, bu, bv):
            def group(t, _):
                def edge(k, sv):
                    e = t * _LANES + k
                    parts = [bu[e, pl.ds(j * _LANES, _LANES)]
                             * bv[e, pl.ds(j * _LANES, _LANES)]
                             for j in range(n_vec)]
                    while len(parts) > 1:
                        parts = [a + b for a, b in zip(parts[::2],
                                                       parts[1::2])]
                    acc = parts[0]
                    for s in (1, 2, 4, 8):
                        acc = acc + _lane_take(acc, lane ^ s)
                    return jnp.where(lane == k, acc, sv)

                sv = lax.fori_loop(0, _LANES, edge,
                                   jnp.zeros((_LANES,), jnp.float32))
                scores[pl.ds(g * _CHUNK + t * _LANES, _LANES)] = sv
                return ()

            lax.fori_loop(0, n_grp, group, ())

        issue(0, rows_u0, rows_v0, sem_u0, sem_v0)

        def pair(h, _):
            g0 = 2 * h
            issue(g0 + 1, rows_u1, rows_v1, sem_u1, sem_v1)
            wait(rows_u0, rows_v0, sem_u0, sem_v0)
            compute(g0, rows_u0, rows_v0)

            @pl.when(h < n_half - 1)
            def _():
                issue(g0 + 2, rows_u0, rows_v0, sem_u0, sem_v0)

            wait(rows_u1, rows_v1, sem_u1, sem_v1)
            compute(g0 + 1, rows_u1, rows_v1)
            return ()

        lax.fori_loop(0, n_half, pair, ())
        pltpu.sync_copy(scores, out_hbm.at[pl.ds(base, e_per_w)])

    mesh = plsc.VectorSubcoreMesh(core_axis_name="c", subcore_axis_name="s",
                                  num_cores=_NUM_CORES,
                                  num_subcores=_NUM_SUBCORES)
    return pl.kernel(
        body,
        out_type=jax.ShapeDtypeStruct((e_pad,), jnp.float32),
        mesh=mesh,
        compiler_params=pltpu.CompilerParams(use_tc_tiling_on_sc=False),
        scratch_types=[
            pltpu.VMEM((e_per_w,), jnp.int32),
            pltpu.VMEM((e_per_w,), jnp.int32),
            pltpu.VMEM((_CHUNK, d), jnp.float32),
            pltpu.VMEM((_CHUNK, d), jnp.float32),
            pltpu.VMEM((_CHUNK, d), jnp.float32),
            pltpu.VMEM((_CHUNK, d), jnp.float32),
            pltpu.VMEM((e_per_w,), jnp.float32),
            pltpu.SemaphoreType.DMA,
            pltpu.SemaphoreType.DMA,
            pltpu.SemaphoreType.DMA,
            pltpu.SemaphoreType.DMA,
        ],
        interpret=interpret,
    )(x, src, dst)


def kernel(x, edge_index):
    e = edge_index.shape[1]
    quantum = _NUM_WORKERS * _CHUNK * 2
    e_pad = ((e + quantum - 1) // quantum) * quantum
    src = edge_index[0].astype(jnp.int32)
    dst = edge_index[1].astype(jnp.int32)
    if e_pad != e:
        pad = jnp.zeros((e_pad - e,), jnp.int32)
        src = jnp.concatenate([src, pad])
        dst = jnp.concatenate([dst, pad])
    return _scores(x, src, dst)[:e, None]


# speedup vs baseline: 3.4397x; 2.0481x over previous
import functools

import jax
import jax.numpy as jnp
from jax import lax
from jax.experimental import pallas as pl
from jax.experimental.pallas import tpu as pltpu
from jax.experimental.pallas import tpu_sc as plsc

_NUM_CORES = 2
_NUM_SUBCORES = 16
_NUM_WORKERS = _NUM_CORES * _NUM_SUBCORES
_LANES = 16
_CHUNK = 80
_SLOW_FRAC = 0.235


def _lane_take(v, idx):
    return lax.gather(
        v, idx[:, None],
        dimension_numbers=lax.GatherDimensionNumbers(
            offset_dims=(), collapsed_slice_dims=(0,), start_index_map=(0,)),
        slice_sizes=(1,),
        mode=lax.GatherScatterMode.PROMISE_IN_BOUNDS)


@functools.partial(jax.jit, static_argnames=("interpret",))
def _scores(x, src, dst, interpret=False):
    e_pad = src.shape[0]
    d = x.shape[1]
    n_vec = d // _LANES
    unit = 2 * _CHUNK
    n_units = e_pad // (_NUM_SUBCORES * unit)
    units_slow = max(1, round(_SLOW_FRAC * n_units))
    units_fast = n_units - units_slow
    e_fast = units_fast * unit
    e_slow = units_slow * unit
    e_max = max(e_fast, e_slow)
    split = _NUM_SUBCORES * e_fast

    def body(x_hbm, src_hbm, dst_hbm, out_hbm,
             idx_u, idx_v, rows_u0, rows_v0, rows_u1, rows_v1, scores,
             sem_u0, sem_v0, sem_u1, sem_v1):
        cid = lax.axis_index("c")
        sid = lax.axis_index("s")
        lane = lax.broadcasted_iota(jnp.int32, (_LANES,), 0)

        def issue(g, bu, bv, su, sv):
            pltpu.async_copy(x_hbm.at[idx_u.at[pl.ds(g * _CHUNK, _CHUNK)]],
                             bu, su)
            pltpu.async_copy(x_hbm.at[idx_v.at[pl.ds(g * _CHUNK, _CHUNK)]],
                             bv, sv)

        def wait(bu, bv, su, sv):
            pltpu.make_async_copy(
                x_hbm.at[idx_u.at[pl.ds(0, _CHUNK)]], bu, su).wait()
            pltpu.make_async_copy(
                x_hbm.at[idx_v.at[pl.ds(0, _CHUNK)]], bv, sv).wait()

        def compute(g, bu, bv):
            def group(t, _):
                def edge(k, sv):
                    e = t * _LANES + k
                    parts = [bu[e, pl.ds(j * _LANES, _LANES)]
                             * bv[e, pl.ds(j * _LANES, _LANES)]
                             for j in range(n_vec)]
                    while len(parts) > 1:
                        parts = [a + b for a, b in zip(parts[::2],
                                                       parts[1::2])]
                    acc = parts[0]
                    for s in (1, 2, 4, 8):
                        acc = acc + _lane_take(acc, lane ^ s)
                    return jnp.where(lane == k, acc, sv)

                sv = lax.fori_loop(0, _LANES, edge,
                                   jnp.zeros((_LANES,), jnp.float32))
                scores[pl.ds(g * _CHUNK + t * _LANES, _LANES)] = sv
                return ()

            lax.fori_loop(0, _CHUNK // _LANES, group, ())

        def run(base, e_tile):
            pltpu.sync_copy(src_hbm.at[pl.ds(base, e_tile)],
                            idx_u.at[pl.ds(0, e_tile)])
            pltpu.sync_copy(dst_hbm.at[pl.ds(base, e_tile)],
                            idx_v.at[pl.ds(0, e_tile)])
            issue(0, rows_u0, rows_v0, sem_u0, sem_v0)

            def pair(h, _):
                g0 = 2 * h
                issue(g0 + 1, rows_u1, rows_v1, sem_u1, sem_v1)
                wait(rows_u0, rows_v0, sem_u0, sem_v0)
                compute(g0, rows_u0, rows_v0)

                @pl.when(h < e_tile // unit - 1)
                def _():
                    issue(g0 + 2, rows_u0, rows_v0, sem_u0, sem_v0)

                wait(rows_u1, rows_v1, sem_u1, sem_v1)
                compute(g0 + 1, rows_u1, rows_v1)
                return ()

            lax.fori_loop(0, e_tile // unit, pair, ())
            pltpu.sync_copy(scores.at[pl.ds(0, e_tile)],
                            out_hbm.at[pl.ds(base, e_tile)])

        @pl.when(cid == 0)
        def _():
            run(sid * e_fast, e_fast)

        @pl.when(cid == 1)
        def _():
            run(split + sid * e_slow, e_slow)

    mesh = plsc.VectorSubcoreMesh(core_axis_name="c", subcore_axis_name="s",
                                  num_cores=_NUM_CORES,
                                  num_subcores=_NUM_SUBCORES)
    return pl.kernel(
        body,
        out_type=jax.ShapeDtypeStruct((e_pad,), jnp.float32),
        mesh=mesh,
        scratch_types=[
            pltpu.VMEM((e_max,), jnp.int32),
            pltpu.VMEM((e_max,), jnp.int32),
            pltpu.VMEM((_CHUNK, d), jnp.float32),
            pltpu.VMEM((_CHUNK, d), jnp.float32),
            pltpu.VMEM((_CHUNK, d), jnp.float32),
            pltpu.VMEM((_CHUNK, d), jnp.float32),
            pltpu.VMEM((e_max,), jnp.float32),
            pltpu.SemaphoreType.DMA,
            pltpu.SemaphoreType.DMA,
            pltpu.SemaphoreType.DMA,
            pltpu.SemaphoreType.DMA,
        ],
        interpret=interpret,
    )(x, src, dst)


def kernel(x, edge_index):
    e = edge_index.shape[1]
    quantum = _NUM_SUBCORES * _CHUNK * 2
    e_pad = ((e + quantum - 1) // quantum) * quantum
    src = edge_index[0].astype(jnp.int32)
    dst = edge_index[1].astype(jnp.int32)
    if e_pad != e:
        pad = jnp.zeros((e_pad - e,), jnp.int32)
        src = jnp.concatenate([src, pad])
        dst = jnp.concatenate([dst, pad])
    return _scores(x, src, dst)[:e, None]
